# initial kernel scaffold (unmeasured)
import jax
import jax.numpy as jnp
from jax import lax
from jax.experimental import pallas as pl
from jax.experimental.pallas import tpu as pltpu

N_DEV = 16


def kernel(x, w_mat):
    m_per, k = x.shape
    _, n = w_mat.shape
    n_per = n // N_DEV

    def body(x_ref, w_ref, out_ref, y_ref, send_sems, recv_sems):
        my = lax.axis_index("i")
        x_val = x_ref[:, :]

        def silu(y):
            return y * jax.nn.sigmoid(y)

        w_own = w_ref[:, pl.ds(my * n_per, n_per)]
        y_own = silu(jnp.dot(x_val, w_own, preferred_element_type=jnp.float32))
        out_ref[pl.ds(my * m_per, m_per), :] = y_own

        rdmas = []
        for j in range(1, N_DEV):
            t = lax.rem(my + j, N_DEV)
            w_j = w_ref[:, pl.ds(t * n_per, n_per)]
            y_ref[j] = silu(
                jnp.dot(x_val, w_j, preferred_element_type=jnp.float32)
            )
            rdma = pltpu.make_async_remote_copy(
                src_ref=y_ref.at[j],
                dst_ref=out_ref.at[pl.ds(my * m_per, m_per), :],
                send_sem=send_sems.at[j - 1],
                recv_sem=recv_sems.at[j - 1],
                device_id=(t,),
                device_id_type=pl.DeviceIdType.MESH,
            )
            rdma.start()
            rdmas.append(rdma)

        for rdma in rdmas:
            rdma.wait()

    return pl.pallas_call(
        body,
        out_shape=jax.ShapeDtypeStruct((N_DEV * m_per, n_per), jnp.float32),
        in_specs=[
            pl.BlockSpec(memory_space=pltpu.VMEM),
            pl.BlockSpec(memory_space=pltpu.VMEM),
        ],
        out_specs=pl.BlockSpec(memory_space=pltpu.VMEM),
        scratch_shapes=[
            pltpu.VMEM((N_DEV, m_per, n_per), jnp.float32),
            pltpu.SemaphoreType.DMA((N_DEV - 1,)),
            pltpu.SemaphoreType.DMA((N_DEV - 1,)),
        ],
        compiler_params=pltpu.CompilerParams(collective_id=0),
    )(x, w_mat)


# baseline (device time: 53627 ns/iter reference)
import jax
import jax.numpy as jnp
from jax import lax
from jax.experimental import pallas as pl
from jax.experimental.pallas import tpu as pltpu

N_DEV = 16


def kernel(x, w_mat):
    m_per, k = x.shape
    _, n = w_mat.shape
    n_per = n // N_DEV

    def body(x_ref, w_ref, out_ref, y_ref, send_sems, recv_sems):
        my = lax.axis_index("i")
        x_val = x_ref[:, :]

        def silu(y):
            return y * jax.nn.sigmoid(y)

        w_own = w_ref[:, pl.ds(my * n_per, n_per)]
        y_own = silu(jnp.dot(x_val, w_own, preferred_element_type=jnp.float32))
        out_ref[pl.ds(my * m_per, m_per), :] = y_own

        rdmas = []
        for j in range(1, N_DEV):
            t = lax.rem(my + j, N_DEV)
            w_j = w_ref[:, pl.ds(t * n_per, n_per)]
            y_ref[j] = silu(
                jnp.dot(x_val, w_j, preferred_element_type=jnp.float32)
            )
            rdma = pltpu.make_async_remote_copy(
                src_ref=y_ref.at[j],
                dst_ref=out_ref.at[pl.ds(my * m_per, m_per), :],
                send_sem=send_sems.at[j - 1],
                recv_sem=recv_sems.at[j - 1],
                device_id=(t,),
                device_id_type=pl.DeviceIdType.MESH,
            )
            rdma.start()
            rdmas.append(rdma)

        for rdma in rdmas:
            rdma.wait()

    return pl.pallas_call(
        body,
        out_shape=jax.ShapeDtypeStruct((N_DEV * m_per, n_per), jnp.float32),
        in_specs=[
            pl.BlockSpec(memory_space=pltpu.VMEM),
            pl.BlockSpec(memory_space=pltpu.VMEM),
        ],
        out_specs=pl.BlockSpec(memory_space=pltpu.VMEM),
        scratch_shapes=[
            pltpu.VMEM((N_DEV, m_per, n_per), jnp.float32),
            pltpu.SemaphoreType.DMA((N_DEV - 1,)),
            pltpu.SemaphoreType.DMA((N_DEV - 1,)),
        ],
        compiler_params=pltpu.CompilerParams(
            vmem_limit_bytes=100 * 1024 * 1024,
        ),
    )(x, w_mat)


# device time: 43709 ns/iter; 1.2269x vs baseline; 1.2269x over previous
import jax
import jax.numpy as jnp
from jax import lax
from jax.experimental import pallas as pl
from jax.experimental.pallas import tpu as pltpu

N_DEV = 16
N_PLANES = 4
PLANE = 4


def kernel(x, w_mat):
    m_per, k_dim = x.shape
    _, n = w_mat.shape
    n_per = n // N_DEV
    n_sb = n // N_PLANES

    def body(x_ref, w_hbm, out_ref, w_ref, y_ref, copy_sems,
             send_sems, recv_sems):
        my = lax.axis_index("i")
        my_z = my // PLANE
        my_p = lax.rem(my, PLANE)

        copies = []
        for c in range(N_PLANES):
            cp = pltpu.make_async_copy(
                w_hbm.at[:, pl.ds(c * n_sb, n_sb)],
                w_ref.at[:, pl.ds(c * n_sb, n_sb)],
                copy_sems.at[c],
            )
            cp.start()
            copies.append(cp)

        bar = pltpu.get_barrier_semaphore()
        for j in range(1, N_DEV):
            pl.semaphore_signal(
                bar, inc=1,
                device_id=(lax.rem(my + j, N_DEV),),
                device_id_type=pl.DeviceIdType.MESH,
            )

        x_val = x_ref[:, :]
        for cp in copies:
            cp.wait()
        pl.semaphore_wait(bar, N_DEV - 1)

        def silu(v):
            return v * jax.nn.sigmoid(v)

        rdmas = []
        for k in range(N_PLANES):
            p = lax.rem(my_z + k, N_PLANES)
            w_sb = w_ref[:, pl.ds(p * n_sb, n_sb)]
            yy = silu(jnp.dot(x_val, w_sb, preferred_element_type=jnp.float32))
            y_ref[k] = yy
            for u in range(PLANE):
                s = PLANE * k + u
                cc = lax.rem(my_p + u, PLANE)
                t = p * PLANE + cc
                if k == 0 and u == 0:
                    out_ref[pl.ds(my * m_per, m_per), :] = (
                        y_ref[k, :, pl.ds(cc * n_per, n_per)])
                    continue
                rdma = pltpu.make_async_remote_copy(
                    src_ref=y_ref.at[k].at[:, pl.ds(cc * n_per, n_per)],
                    dst_ref=out_ref.at[pl.ds(my * m_per, m_per), :],
                    send_sem=send_sems.at[s],
                    recv_sem=recv_sems.at[s],
                    device_id=(t,),
                    device_id_type=pl.DeviceIdType.MESH,
                )
                rdma.start()
                rdmas.append(rdma)

        for rdma in rdmas:
            rdma.wait()

    return pl.pallas_call(
        body,
        out_shape=jax.ShapeDtypeStruct((N_DEV * m_per, n_per), jnp.float32),
        in_specs=[
            pl.BlockSpec(memory_space=pltpu.VMEM),
            pl.BlockSpec(memory_space=pltpu.MemorySpace.HBM),
        ],
        out_specs=pl.BlockSpec(memory_space=pltpu.VMEM),
        scratch_shapes=[
            pltpu.VMEM((k_dim, n), jnp.float32),
            pltpu.VMEM((N_PLANES, m_per, n_sb), jnp.float32),
            pltpu.SemaphoreType.DMA((N_PLANES,)),
            pltpu.SemaphoreType.DMA((N_DEV,)),
            pltpu.SemaphoreType.DMA((N_DEV,)),
        ],
        compiler_params=pltpu.CompilerParams(
            vmem_limit_bytes=100 * 1024 * 1024,
            collective_id=0,
        ),
    )(x, w_mat)
